# baseline (speedup 1.0000x reference)
import functools

import jax
import jax.numpy as jnp
from jax import lax
from jax.experimental import pallas as pl
from jax.experimental.pallas import tpu as pltpu
from jax.experimental.pallas import tpu_sc as plsc

N_NODES = 10000
N_EDGES = 320000
D = 128
EPS = 1e-5

NC = 2
NS = 16
NW = NC * NS
CHUNK = 128
GSZ = 8
NG = 10
CPT = GSZ * NG
E_PAD = NW * CPT * CHUNK
N_ACC = 10016
DUMMY = 10000
RPT = 624
RSPAN = 640
DROWS = 80
LANES = 16


def _sc_segment_sum(feature, src_r, dst_r):
    mesh = plsc.VectorSubcoreMesh(core_axis_name="c", subcore_axis_name="s",
                                  num_cores=NC, num_subcores=NS)

    @functools.partial(
        pl.kernel,
        out_type=(
            jax.ShapeDtypeStruct((NC, N_NODES, D), jnp.float32),
            jax.ShapeDtypeStruct((NW, DROWS, D), jnp.float32),
        ),
        mesh=mesh,
        scratch_types=[
            pltpu.VMEM_SHARED((N_ACC, D), jnp.float32),
            pltpu.VMEM((2, GSZ, CHUNK), jnp.int32),
            pltpu.VMEM((2, GSZ, CHUNK), jnp.int32),
            pltpu.VMEM((CHUNK, D), jnp.float32),
            pltpu.VMEM((CHUNK, D), jnp.float32),
            pltpu.VMEM((DROWS, D), jnp.float32),
            pltpu.SemaphoreType.DMA,
            pltpu.SemaphoreType.DMA,
        ],
        compiler_params=pltpu.CompilerParams(needs_layout_passes=False),
    )
    def k(feat_hbm, src_hbm, dst_hbm, s_out, deg_out,
          acc_sh, sidx, didx, rows0, rows1, deg2d, gsem0, gsem1):
        cid = lax.axis_index("c")
        sid = lax.axis_index("s")
        slab = cid * NS + sid

        zv = jnp.zeros((LANES,), jnp.float32)

        def fill_body(i, _):
            for kk in range(D // LANES):
                rows0[i, pl.ds(kk * LANES, LANES)] = zv
            return 0

        lax.fori_loop(0, CHUNK, fill_body, 0)

        def dfill_body(i, _):
            for kk in range(D // LANES):
                deg2d[i, pl.ds(kk * LANES, LANES)] = zv
            return 0

        lax.fori_loop(0, DROWS, dfill_body, 0)

        for r in range(5):
            pltpu.sync_copy(rows0, acc_sh.at[pl.ds(sid * RPT + r * CHUNK,
                                                   CHUNK)])

        @pl.when(sid == 0)
        def _():
            pltpu.sync_copy(rows0.at[pl.ds(0, N_ACC - N_NODES)],
                            acc_sh.at[pl.ds(N_NODES, N_ACC - N_NODES)])

        plsc.subcore_barrier()

        def load_group(g, b):
            pltpu.sync_copy(src_hbm.at[slab, pl.ds(g * GSZ, GSZ)],
                            sidx.at[b])
            pltpu.sync_copy(dst_hbm.at[slab, pl.ds(g * GSZ, GSZ)],
                            didx.at[b])

        def hist(gb, j):
            for kk in range(CHUNK // LANES):
                dv = didx[gb, j, pl.ds(kk * LANES, LANES)]
                cnt, last = plsc.scan_count(dv)
                plsc.addupdate_scatter(
                    deg2d,
                    [lax.shift_right_logical(dv, 7),
                     jnp.bitwise_and(dv, 127)],
                    cnt.astype(jnp.float32), mask=last)

        load_group(0, 0)
        pltpu.async_copy(feat_hbm.at[sidx.at[0, 0]], rows0, gsem0)

        def group_body(g, _):
            gb = jnp.bitwise_and(g, 1)
            nb = 1 - gb

            @pl.when(g < NG - 1)
            def _():
                load_group(g + 1, nb)

            xb = jnp.where(g < NG - 1, nb, gb)

            for q in range(GSZ // 2):
                j0, j1 = 2 * q, 2 * q + 1
                pltpu.make_async_copy(feat_hbm.at[sidx.at[gb, j0]], rows0,
                                      gsem0).wait()
                pltpu.async_copy(feat_hbm.at[sidx.at[gb, j1]], rows1, gsem1)
                pltpu.sync_copy(rows0, acc_sh.at[didx.at[gb, j0]], add=True)
                hist(gb, j0)
                pltpu.make_async_copy(feat_hbm.at[sidx.at[gb, j1]], rows1,
                                      gsem1).wait()
                if q < GSZ // 2 - 1:
                    pltpu.async_copy(feat_hbm.at[sidx.at[gb, j1 + 1]],
                                     rows0, gsem0)
                else:
                    pltpu.async_copy(feat_hbm.at[sidx.at[xb, 0]],
                                     rows0, gsem0)
                pltpu.sync_copy(rows1, acc_sh.at[didx.at[gb, j1]], add=True)
                hist(gb, j1)
            return 0

        lax.fori_loop(0, NG, group_body, 0)

        pltpu.make_async_copy(feat_hbm.at[sidx.at[0, 0]], rows0,
                              gsem0).wait()

        plsc.subcore_barrier()

        pltpu.sync_copy(acc_sh.at[pl.ds(sid * RPT, RSPAN)],
                        s_out.at[cid, pl.ds(sid * RPT, RSPAN)])
        pltpu.sync_copy(deg2d, deg_out.at[slab])

    return k(feature, src_r, dst_r)


def _tc_body(feat_ref, s_ref, deg_ref, snorm_ref, w_ref, b_ref, g_ref,
             be_ref, out_ref):
    f = feat_ref[...]
    s = s_ref[0] + s_ref[1]
    deg = jnp.sum(deg_ref[...], axis=1, keepdims=True)
    agg = jnp.where(deg > 0.0, s / jnp.maximum(deg, 1.0), f)
    h = lax.dot_general(agg, w_ref[...], (((1,), (1,)), ((), ())),
                        preferred_element_type=jnp.float32)
    h = (h + b_ref[...]) * snorm_ref[...]
    mean = jnp.mean(h, axis=0, keepdims=True)
    var = jnp.mean((h - mean) ** 2, axis=0, keepdims=True)
    h = (h - mean) * lax.rsqrt(var + EPS) * g_ref[...] + be_ref[...]
    out_ref[...] = f + jnp.maximum(h, 0.0)


def kernel(feature, edge_index, snorm_n, W, b, gamma, beta):
    npad = E_PAD - N_EDGES
    src_r = jnp.concatenate(
        [edge_index[0], jnp.zeros((npad,), jnp.int32)]).reshape(
            NW, CPT * CHUNK)
    pad_dst = DUMMY + jnp.arange(npad, dtype=jnp.int32) % (N_ACC - N_NODES)
    dst_r = jnp.concatenate([edge_index[1], pad_dst]).reshape(
        NW, CPT * CHUNK)
    src_r = src_r.reshape(NW, CPT, CHUNK)
    dst_r = dst_r.reshape(NW, CPT, CHUNK)
    s_part, deg_hist = _sc_segment_sum(feature, src_r, dst_r)
    deg_t = deg_hist.reshape(NW, DROWS * D)[:, :N_NODES].T
    out = pl.pallas_call(
        _tc_body,
        out_shape=jax.ShapeDtypeStruct((N_NODES, D), jnp.float32),
        compiler_params=pltpu.CompilerParams(
            vmem_limit_bytes=100 * 1024 * 1024),
    )(feature, s_part, deg_t, snorm_n,
      W, b.reshape(1, D), gamma.reshape(1, D), beta.reshape(1, D))
    return out


# speedup vs baseline: 1.3765x; 1.3765x over previous
import functools

import jax
import jax.numpy as jnp
from jax import lax
from jax.experimental import pallas as pl
from jax.experimental.pallas import tpu as pltpu
from jax.experimental.pallas import tpu_sc as plsc

N_NODES = 10000
N_EDGES = 320000
D = 128
EPS = 1e-5

NC = 2
NS = 16
NW = NC * NS
CHUNK = 128
CPT = 79
E_PAD = NW * CPT * CHUNK
N_ACC = 10016
DUMMY = 10000
RPT = 624
RSPAN = 640
DROWS = 80
LANES = 16


def _sc_segment_sum(feature, src_flat, dst_flat):
    mesh = plsc.VectorSubcoreMesh(core_axis_name="c", subcore_axis_name="s",
                                  num_cores=NC, num_subcores=NS)

    @functools.partial(
        pl.kernel,
        out_type=(
            jax.ShapeDtypeStruct((NC, N_NODES, D), jnp.float32),
            jax.ShapeDtypeStruct((NW, DROWS, D), jnp.float32),
        ),
        mesh=mesh,
        scratch_types=[
            pltpu.VMEM_SHARED((N_ACC, D), jnp.float32),
            pltpu.VMEM((2, CHUNK), jnp.int32),
            pltpu.VMEM((2, CHUNK), jnp.int32),
            pltpu.VMEM((CHUNK, D), jnp.float32),
            pltpu.VMEM((CHUNK, D), jnp.float32),
            pltpu.VMEM((DROWS, D), jnp.float32),
            pltpu.SemaphoreType.DMA,
            pltpu.SemaphoreType.DMA,
        ],
        compiler_params=pltpu.CompilerParams(needs_layout_passes=False),
    )
    def k(feat_hbm, src_hbm, dst_hbm, s_out, deg_out,
          acc_sh, sidx, didx, rows0, rows1, deg2d, gsem0, gsem1):
        cid = lax.axis_index("c")
        sid = lax.axis_index("s")
        slab = cid * NS + sid

        zv = jnp.zeros((LANES,), jnp.float32)

        def fill_body(i, _):
            for kk in range(D // LANES):
                rows0[i, pl.ds(kk * LANES, LANES)] = zv
            return 0

        lax.fori_loop(0, CHUNK, fill_body, 0)

        def dfill_body(i, _):
            for kk in range(D // LANES):
                deg2d[i, pl.ds(kk * LANES, LANES)] = zv
            return 0

        lax.fori_loop(0, DROWS, dfill_body, 0)

        for r in range(5):
            pltpu.sync_copy(rows0, acc_sh.at[pl.ds(sid * RPT + r * CHUNK,
                                                   CHUNK)])

        @pl.when(sid == 0)
        def _():
            pltpu.sync_copy(rows0.at[pl.ds(0, N_ACC - N_NODES)],
                            acc_sh.at[pl.ds(N_NODES, N_ACC - N_NODES)])

        plsc.subcore_barrier()

        def load_idx(c, b):
            base = (slab * CPT + c) * CHUNK
            pltpu.sync_copy(src_hbm.at[pl.ds(base, CHUNK)], sidx.at[b])
            pltpu.sync_copy(dst_hbm.at[pl.ds(base, CHUNK)], didx.at[b])

        def hist(b):
            for kk in range(CHUNK // LANES):
                dv = didx[b, pl.ds(kk * LANES, LANES)]
                cnt, last = plsc.scan_count(dv)
                plsc.addupdate_scatter(
                    deg2d,
                    [lax.shift_right_logical(dv, 7),
                     jnp.bitwise_and(dv, 127)],
                    cnt.astype(jnp.float32), mask=last)

        load_idx(0, 0)
        pltpu.async_copy(feat_hbm.at[sidx.at[0]], rows0, gsem0)

        def pair_body(p, _):
            c0 = 2 * p
            load_idx(c0 + 1, 1)
            pltpu.make_async_copy(feat_hbm.at[sidx.at[0]], rows0,
                                  gsem0).wait()
            pltpu.async_copy(feat_hbm.at[sidx.at[1]], rows1, gsem1)
            pltpu.sync_copy(rows0, acc_sh.at[didx.at[0]], add=True)
            hist(0)
            load_idx(c0 + 2, 0)
            pltpu.make_async_copy(feat_hbm.at[sidx.at[1]], rows1,
                                  gsem1).wait()
            pltpu.async_copy(feat_hbm.at[sidx.at[0]], rows0, gsem0)
            pltpu.sync_copy(rows1, acc_sh.at[didx.at[1]], add=True)
            hist(1)
            return 0

        lax.fori_loop(0, (CPT - 1) // 2, pair_body, 0)

        pltpu.make_async_copy(feat_hbm.at[sidx.at[0]], rows0, gsem0).wait()
        pltpu.sync_copy(rows0, acc_sh.at[didx.at[0]], add=True)
        hist(0)

        plsc.subcore_barrier()

        pltpu.sync_copy(acc_sh.at[pl.ds(sid * RPT, RSPAN)],
                        s_out.at[cid, pl.ds(sid * RPT, RSPAN)])
        pltpu.sync_copy(deg2d, deg_out.at[slab])

    return k(feature, src_flat, dst_flat)


def _tc_body(feat_ref, s_ref, deg_ref, snorm_ref, w_ref, b_ref, g_ref,
             be_ref, out_ref):
    f = feat_ref[...]
    s = s_ref[0] + s_ref[1]
    deg = jnp.sum(deg_ref[...], axis=1, keepdims=True)
    agg = jnp.where(deg > 0.0, s / jnp.maximum(deg, 1.0), f)
    h = lax.dot_general(agg, w_ref[...], (((1,), (1,)), ((), ())),
                        preferred_element_type=jnp.float32)
    h = (h + b_ref[...]) * snorm_ref[...]
    mean = jnp.mean(h, axis=0, keepdims=True)
    var = jnp.mean((h - mean) ** 2, axis=0, keepdims=True)
    h = (h - mean) * lax.rsqrt(var + EPS) * g_ref[...] + be_ref[...]
    out_ref[...] = f + jnp.maximum(h, 0.0)


def kernel(feature, edge_index, snorm_n, W, b, gamma, beta):
    ppt = E_PAD // NW - N_EDGES // NW
    pad_src = jnp.zeros((NW, ppt), jnp.int32)
    pad_dst = DUMMY + (jnp.arange(ppt, dtype=jnp.int32) %
                       (N_ACC - N_NODES))[None, :] + pad_src
    src_flat = jnp.concatenate(
        [edge_index[0].reshape(NW, N_EDGES // NW), pad_src],
        axis=1).reshape(-1)
    dst_flat = jnp.concatenate(
        [edge_index[1].reshape(NW, N_EDGES // NW), pad_dst],
        axis=1).reshape(-1)
    s_part, deg_hist = _sc_segment_sum(feature, src_flat, dst_flat)
    deg_t = deg_hist.reshape(NW, DROWS * D)[:, :N_NODES].T
    out = pl.pallas_call(
        _tc_body,
        out_shape=jax.ShapeDtypeStruct((N_NODES, D), jnp.float32),
        compiler_params=pltpu.CompilerParams(
            vmem_limit_bytes=100 * 1024 * 1024),
    )(feature, s_part, deg_t, snorm_n,
      W, b.reshape(1, D), gamma.reshape(1, D), beta.reshape(1, D))
    return out


# speedup vs baseline: 1.5075x; 1.0952x over previous
import functools

import jax
import jax.numpy as jnp
from jax import lax
from jax.experimental import pallas as pl
from jax.experimental.pallas import tpu as pltpu
from jax.experimental.pallas import tpu_sc as plsc

N_NODES = 10000
N_EDGES = 320000
D = 128
EPS = 1e-5

NC = 2
NS = 16
NW = NC * NS
CHUNK = 128
CPT = 79
E_PAD = NW * CPT * CHUNK
N_ACC = 10016
DUMMY = 10000
RPT = 624
RSPAN = 640
DROWS = 80
LANES = 16


def _sc_segment_sum(feature, src_flat, dst_flat):
    mesh = plsc.VectorSubcoreMesh(core_axis_name="c", subcore_axis_name="s",
                                  num_cores=NC, num_subcores=NS)

    @functools.partial(
        pl.kernel,
        out_type=(
            jax.ShapeDtypeStruct((NC, N_NODES, D), jnp.float32),
            jax.ShapeDtypeStruct((NW, DROWS, D), jnp.float32),
        ),
        mesh=mesh,
        scratch_types=[
            pltpu.VMEM_SHARED((N_ACC, D), jnp.float32),
            pltpu.VMEM((2, CHUNK), jnp.int32),
            pltpu.VMEM((2, CHUNK), jnp.int32),
            pltpu.VMEM((CHUNK, D), jnp.float32),
            pltpu.VMEM((CHUNK, D), jnp.float32),
            pltpu.VMEM((DROWS, D), jnp.float32),
            pltpu.SemaphoreType.DMA,
            pltpu.SemaphoreType.DMA,
            pltpu.SemaphoreType.DMA,
            pltpu.SemaphoreType.DMA,
        ],
        compiler_params=pltpu.CompilerParams(needs_layout_passes=False),
    )
    def k(feat_hbm, src_hbm, dst_hbm, s_out, deg_out,
          acc_sh, sidx, didx, rows0, rows1, deg2d, gsem0, gsem1,
          isem0, isem1):
        cid = lax.axis_index("c")
        sid = lax.axis_index("s")
        slab = cid * NS + sid

        zv = jnp.zeros((LANES,), jnp.float32)

        def fill_body(i, _):
            for kk in range(D // LANES):
                rows0[i, pl.ds(kk * LANES, LANES)] = zv
            return 0

        lax.fori_loop(0, CHUNK, fill_body, 0)

        def dfill_body(i, _):
            for kk in range(D // LANES):
                deg2d[i, pl.ds(kk * LANES, LANES)] = zv
            return 0

        lax.fori_loop(0, DROWS, dfill_body, 0)

        for r in range(5):
            pltpu.sync_copy(rows0, acc_sh.at[pl.ds(sid * RPT + r * CHUNK,
                                                   CHUNK)])

        @pl.when(sid == 0)
        def _():
            pltpu.sync_copy(rows0.at[pl.ds(0, N_ACC - N_NODES)],
                            acc_sh.at[pl.ds(N_NODES, N_ACC - N_NODES)])

        plsc.subcore_barrier()

        def load_idx(c, b, isem):
            base = (slab * CPT + c) * CHUNK
            pltpu.async_copy(src_hbm.at[pl.ds(base, CHUNK)], sidx.at[b],
                             isem)
            pltpu.async_copy(dst_hbm.at[pl.ds(base, CHUNK)], didx.at[b],
                             isem)

        def wait_idx(c, b, isem):
            base = (slab * CPT + c) * CHUNK
            pltpu.make_async_copy(src_hbm.at[pl.ds(base, CHUNK)],
                                  sidx.at[b], isem).wait()
            pltpu.make_async_copy(dst_hbm.at[pl.ds(base, CHUNK)],
                                  didx.at[b], isem).wait()

        def hist(b):
            for kk in range(CHUNK // LANES):
                dv = didx[b, pl.ds(kk * LANES, LANES)]
                cnt, last = plsc.scan_count(dv)
                plsc.addupdate_scatter(
                    deg2d,
                    [lax.shift_right_logical(dv, 7),
                     jnp.bitwise_and(dv, 127)],
                    cnt.astype(jnp.float32), mask=last)

        load_idx(0, 0, isem0)
        wait_idx(0, 0, isem0)
        pltpu.async_copy(feat_hbm.at[sidx.at[0]], rows0, gsem0)
        load_idx(1, 1, isem1)

        def pair_body(p, _):
            c0 = 2 * p
            pltpu.make_async_copy(feat_hbm.at[sidx.at[0]], rows0,
                                  gsem0).wait()
            wait_idx(c0 + 1, 1, isem1)
            pltpu.async_copy(feat_hbm.at[sidx.at[1]], rows1, gsem1)
            pltpu.sync_copy(rows0, acc_sh.at[didx.at[0]], add=True)
            hist(0)
            load_idx(c0 + 2, 0, isem0)
            pltpu.make_async_copy(feat_hbm.at[sidx.at[1]], rows1,
                                  gsem1).wait()
            wait_idx(c0 + 2, 0, isem0)
            pltpu.async_copy(feat_hbm.at[sidx.at[0]], rows0, gsem0)
            pltpu.sync_copy(rows1, acc_sh.at[didx.at[1]], add=True)
            hist(1)

            @pl.when(p < (CPT - 1) // 2 - 1)
            def _():
                load_idx(c0 + 3, 1, isem1)

            return 0

        lax.fori_loop(0, (CPT - 1) // 2, pair_body, 0)

        pltpu.make_async_copy(feat_hbm.at[sidx.at[0]], rows0, gsem0).wait()
        pltpu.sync_copy(rows0, acc_sh.at[didx.at[0]], add=True)
        hist(0)

        plsc.subcore_barrier()

        pltpu.sync_copy(acc_sh.at[pl.ds(sid * RPT, RSPAN)],
                        s_out.at[cid, pl.ds(sid * RPT, RSPAN)])
        pltpu.sync_copy(deg2d, deg_out.at[slab])

    return k(feature, src_flat, dst_flat)


def _tc_body(feat_ref, s_ref, deg_ref, snorm_ref, w_ref, b_ref, g_ref,
             be_ref, out_ref):
    f = feat_ref[...]
    s = s_ref[0] + s_ref[1]
    deg = jnp.sum(deg_ref[...], axis=1, keepdims=True)
    agg = jnp.where(deg > 0.0, s / jnp.maximum(deg, 1.0), f)
    h = lax.dot_general(agg, w_ref[...], (((1,), (1,)), ((), ())),
                        preferred_element_type=jnp.float32)
    h = (h + b_ref[...]) * snorm_ref[...]
    mean = jnp.mean(h, axis=0, keepdims=True)
    var = jnp.mean((h - mean) ** 2, axis=0, keepdims=True)
    h = (h - mean) * lax.rsqrt(var + EPS) * g_ref[...] + be_ref[...]
    out_ref[...] = f + jnp.maximum(h, 0.0)


def kernel(feature, edge_index, snorm_n, W, b, gamma, beta):
    npad = E_PAD - N_EDGES
    src_flat = jnp.concatenate([edge_index[0],
                                jnp.zeros((npad,), jnp.int32)])
    dst_flat = jnp.concatenate([edge_index[1],
                                jnp.full((npad,), DUMMY, jnp.int32)])
    s_part, deg_hist = _sc_segment_sum(feature, src_flat, dst_flat)
    deg_t = deg_hist.reshape(NW, DROWS * D)[:, :N_NODES].T
    out = pl.pallas_call(
        _tc_body,
        out_shape=jax.ShapeDtypeStruct((N_NODES, D), jnp.float32),
        compiler_params=pltpu.CompilerParams(
            vmem_limit_bytes=100 * 1024 * 1024),
    )(feature, s_part, deg_t, snorm_n,
      W, b.reshape(1, D), gamma.reshape(1, D), beta.reshape(1, D))
    return out
